# Initial kernel scaffold; baseline (speedup 1.0000x reference)
import jax
import jax.numpy as jnp
from jax.experimental import pallas as pl


def kernel(x, pos, Wq, W1k, W2k, W1v, W2v, Wd):
    raise NotImplementedError("write your pallas kernel here")


# speedup vs baseline: 1.0888x; 1.0888x over previous
import math

import jax
import jax.numpy as jnp
from jax.experimental import pallas as pl

_N = 10000
_DIN = 32
_DQ = 16
_DK = 16
_DOUT = 32
_NB = 10
_HID = 16
_R = 0.073
_E = 32 * _N

_BE = 512
_BN = 1000
_C = 1.14136 * (math.e ** 2)


def _soft_unit_step(t):
    safe = jnp.where(t > 0, t, 1.0)
    return jnp.where(t > 0, jnp.exp(-1.0 / safe), 0.0)


def _prep_kernel(x_ref, wqd_ref, qd_ref):
    qd_ref[...] = jnp.dot(x_ref[...], wqd_ref[...],
                          preferred_element_type=jnp.float32)


def _edge_kernel(gs_ref, gd_ref, w1_ref, wbig_ref, out_ref):
    gs = gs_ref[...]
    gd = gd_ref[...]
    xs = gs[:, :32]
    ps = gs[:, 32:35]
    qd = gd[:, :16]
    pd = gd[:, 16:19]

    d = ps - pd
    d2 = jnp.sum(d * d, axis=1, keepdims=True)
    el = jnp.sqrt(d2)
    step = _R / (_NB + 1)
    m = (jax.lax.broadcasted_iota(jnp.int32, (_BE, _NB), 1) + 1).astype(jnp.float32)
    diff = el / step - m
    emb = (_C * (_NB ** 0.5)) * _soft_unit_step(diff + 1.0) * _soft_unit_step(1.0 - diff)
    cutoff = _soft_unit_step(10.0 * (1.0 - el / _R))

    emb16 = jnp.concatenate(
        [emb, jnp.zeros((_BE, _HID - _NB), jnp.float32)], axis=1)
    h = jnp.dot(emb16, w1_ref[...], preferred_element_type=jnp.float32)
    h = h * jax.nn.sigmoid(h)
    hk = h[:, :_HID]
    hv = h[:, _HID:]

    u = jnp.dot(xs, wbig_ref[...], preferred_element_type=jnp.float32)
    k = jnp.zeros((_BE, _DK), jnp.float32)
    v = jnp.zeros((_BE, _DOUT), jnp.float32)
    for hh in range(_HID):
        k = k + hk[:, hh:hh + 1] * u[:, hh * _DK:(hh + 1) * _DK]
        v = v + hv[:, hh:hh + 1] * u[:, _HID * _DK + hh * _DOUT:
                                      _HID * _DK + (hh + 1) * _DOUT]

    logit = jnp.sum(qd * k, axis=1, keepdims=True)
    expv = jnp.where(d2 > 0.0, cutoff * jnp.exp(logit), 0.0)
    sv = jnp.sqrt(expv) * v
    out_ref[...] = jnp.concatenate(
        [sv, expv, jnp.zeros((_BE, 31), jnp.float32)], axis=1)


def _finalize_kernel(parts_ref, out_ref):
    p = jnp.sum(parts_ref[...], axis=0)
    z = p[:, 32:33]
    z = jnp.where(z == 0.0, 1.0, z)
    out_ref[...] = p[:, :32] * jax.lax.rsqrt(z)


def kernel(x, pos, Wq, W1k, W2k, W1v, W2v, Wd):
    sq = jnp.sum(pos * pos, axis=1)
    d2m = sq[:, None] + sq[None, :] - 2.0 * (pos @ pos.T)
    mask = (d2m < _R * _R) & (~jnp.eye(_N, dtype=bool))
    dst, src = jnp.nonzero(mask, size=_E, fill_value=0)
    src = src.astype(jnp.int32)
    dst = dst.astype(jnp.int32)

    w1 = jnp.zeros((_HID, 2 * _HID), jnp.float32)
    w1 = w1.at[:_NB, :_HID].set(W1k / math.sqrt(_NB))
    w1 = w1.at[:_NB, _HID:].set(W1v / math.sqrt(_NB))
    bk = W2k.reshape(_HID, _DIN, _DK).transpose(1, 0, 2).reshape(_DIN, _HID * _DK)
    bv = W2v.reshape(_HID, _DIN, _DOUT).transpose(1, 0, 2).reshape(_DIN, _HID * _DOUT)
    wbig = jnp.concatenate([bk, bv], axis=1) / (math.sqrt(_HID) * math.sqrt(_DIN))
    wqd = (Wq @ Wd) / (math.sqrt(_DIN) * math.sqrt(_DQ * _DK))

    qd = pl.pallas_call(
        _prep_kernel,
        grid=(_N // _BN,),
        in_specs=[
            pl.BlockSpec((_BN, _DIN), lambda b: (b, 0)),
            pl.BlockSpec((_DIN, _DK), lambda b: (0, 0)),
        ],
        out_specs=pl.BlockSpec((_BN, _DK), lambda b: (b, 0)),
        out_shape=jax.ShapeDtypeStruct((_N, _DK), jnp.float32),
    )(x, wqd)

    table_s = jnp.concatenate([x, pos, jnp.zeros((_N, 13), jnp.float32)], axis=1)
    table_d = jnp.concatenate([qd, pos, jnp.zeros((_N, 13), jnp.float32)], axis=1)
    gs = jnp.take(table_s, src, axis=0)
    gd = jnp.take(table_d, dst, axis=0)

    contrib = pl.pallas_call(
        _edge_kernel,
        grid=(_E // _BE,),
        in_specs=[
            pl.BlockSpec((_BE, 48), lambda b: (b, 0)),
            pl.BlockSpec((_BE, 32), lambda b: (b, 0)),
            pl.BlockSpec((_HID, 2 * _HID), lambda b: (0, 0)),
            pl.BlockSpec((_DIN, _HID * (_DK + _DOUT)), lambda b: (0, 0)),
        ],
        out_specs=pl.BlockSpec((_BE, 64), lambda b: (b, 0)),
        out_shape=jax.ShapeDtypeStruct((_E, 64), jnp.float32),
    )(gs, gd, w1, wbig)

    acc = jax.ops.segment_sum(contrib, dst, num_segments=_N)
    parts = acc[None]

    out = pl.pallas_call(
        _finalize_kernel,
        grid=(_N // _BN,),
        in_specs=[pl.BlockSpec((1, _BN, 64), lambda b: (0, b, 0))],
        out_specs=pl.BlockSpec((_BN, _DOUT), lambda b: (b, 0)),
        out_shape=jax.ShapeDtypeStruct((_N, _DOUT), jnp.float32),
    )(parts)
    return out


# speedup vs baseline: 2.3052x; 2.1172x over previous
import functools
import math

import jax
import jax.numpy as jnp
from jax import lax
from jax.experimental import pallas as pl
from jax.experimental.pallas import tpu as pltpu
from jax.experimental.pallas import tpu_sc as plsc

_N = 10000
_DIN = 32
_DQ = 16
_DK = 16
_DOUT = 32
_NB = 10
_HID = 16
_R = 0.073
_NW = 32
_SLOT = 10240
_EP = _NW * _SLOT
_NC = 10240
_BR = 200
_BE = 512
_BN = 1000
_C = 1.14136 * (math.e ** 2)


def _soft_unit_step(t):
    safe = jnp.where(t > 0, t, 1.0)
    return jnp.where(t > 0, jnp.exp(-1.0 / safe), 0.0)


def _mask_kernel(pb_ref, post_ref, mask_ref):
    b = pl.program_id(0)
    pb = pb_ref[...]
    post = post_ref[...]
    sqr = jnp.sum(pb * pb, axis=1, keepdims=True)
    sqc = jnp.sum(post * post, axis=0, keepdims=True)
    d2 = sqr + sqc - 2.0 * jnp.dot(pb, post, preferred_element_type=jnp.float32)
    row = lax.broadcasted_iota(jnp.int32, (_BR, _NC), 0) + b * _BR
    col = lax.broadcasted_iota(jnp.int32, (_BR, _NC), 1)
    ok = (d2 < _R * _R) & (col != row) & (col < _N)
    mask_ref[...] = jnp.where(ok, 1.0, 0.0)


def _compact_kernel(maskf, src_out, dst_out, row_v, sidx, sdst, sem):
    w = lax.axis_index("s") * 2 + lax.axis_index("c")
    lanes = lax.iota(jnp.int32, 16)
    zeros16 = jnp.zeros((16,), jnp.int32)

    def zero_body(i, _):
        sidx[pl.ds(i * 16, 16)] = zeros16
        sdst[pl.ds(i * 16, 16)] = zeros16
        return 0

    lax.fori_loop(0, _SLOT // 16, zero_body, 0)

    def row_body(i, fill):
        r = w + _NW * i

        def do_row(fill):
            pltpu.async_copy(maskf.at[r], row_v, sem).wait()

            def chunk_body(c, fill):
                mv = row_v[pl.ds(c * 16, 16)]
                mi = mv.astype(jnp.int32)
                cnt = jnp.sum(mi)

                def write(fill):
                    cs = plsc.cumsum(mi)
                    pos = jnp.minimum(fill + cs - 1, _SLOT - 1)
                    mok = mv != 0.0
                    colv = c * 16 + lanes
                    plsc.store_scatter(sidx, [pos], colv, mask=mok)
                    plsc.store_scatter(
                        sdst, [pos], jnp.full((16,), r, jnp.int32), mask=mok)
                    return fill + cnt

                return lax.cond(cnt > 0, write, lambda f: f, fill)

            return lax.fori_loop(0, _NC // 16, chunk_body, fill)

        return lax.cond(r < _N, do_row, lambda f: f, fill)

    lax.fori_loop(0, (_N + _NW - 1) // _NW, row_body, jnp.int32(0))

    pltpu.sync_copy(sidx, src_out.at[pl.ds(w * _SLOT, _SLOT)])
    pltpu.sync_copy(sdst, dst_out.at[pl.ds(w * _SLOT, _SLOT)])


def _prep_kernel(x_ref, wqd_ref, qd_ref):
    qd_ref[...] = jnp.dot(x_ref[...], wqd_ref[...],
                          preferred_element_type=jnp.float32)


def _edge_kernel(gs_ref, gd_ref, w1_ref, wbig_ref, out_ref):
    gs = gs_ref[...]
    gd = gd_ref[...]
    xs = gs[:, :32]
    ps = gs[:, 32:35]
    qd = gd[:, :16]
    pd = gd[:, 16:19]

    d = ps - pd
    d2 = jnp.sum(d * d, axis=1, keepdims=True)
    el = jnp.sqrt(d2)
    step = _R / (_NB + 1)
    m = (lax.broadcasted_iota(jnp.int32, (_BE, _NB), 1) + 1).astype(jnp.float32)
    diff = el / step - m
    emb = (_C * (_NB ** 0.5)) * _soft_unit_step(diff + 1.0) * _soft_unit_step(1.0 - diff)
    cutoff = _soft_unit_step(10.0 * (1.0 - el / _R))

    emb16 = jnp.concatenate(
        [emb, jnp.zeros((_BE, _HID - _NB), jnp.float32)], axis=1)
    h = jnp.dot(emb16, w1_ref[...], preferred_element_type=jnp.float32)
    h = h * jax.nn.sigmoid(h)
    hk = h[:, :_HID]
    hv = h[:, _HID:]

    u = jnp.dot(xs, wbig_ref[...], preferred_element_type=jnp.float32)
    k = jnp.zeros((_BE, _DK), jnp.float32)
    v = jnp.zeros((_BE, _DOUT), jnp.float32)
    for hh in range(_HID):
        k = k + hk[:, hh:hh + 1] * u[:, hh * _DK:(hh + 1) * _DK]
        v = v + hv[:, hh:hh + 1] * u[:, _HID * _DK + hh * _DOUT:
                                      _HID * _DK + (hh + 1) * _DOUT]

    logit = jnp.sum(qd * k, axis=1, keepdims=True)
    expv = jnp.where(d2 > 0.0, cutoff * jnp.exp(logit), 0.0)
    sv = jnp.sqrt(expv) * v
    out_ref[...] = jnp.concatenate(
        [sv, expv, jnp.zeros((_BE, 31), jnp.float32)], axis=1)


def _finalize_kernel(parts_ref, out_ref):
    p = jnp.sum(parts_ref[...], axis=0)
    z = p[:, 32:33]
    z = jnp.where(z == 0.0, 1.0, z)
    out_ref[...] = p[:, :32] * jax.lax.rsqrt(z)


def kernel(x, pos, Wq, W1k, W2k, W1v, W2v, Wd):
    post = jnp.concatenate(
        [pos.T, jnp.zeros((3, _NC - _N), jnp.float32)], axis=1)
    maskf = pl.pallas_call(
        _mask_kernel,
        grid=(_N // _BR,),
        in_specs=[
            pl.BlockSpec((_BR, 3), lambda b: (b, 0)),
            pl.BlockSpec((3, _NC), lambda b: (0, 0)),
        ],
        out_specs=pl.BlockSpec((_BR, _NC), lambda b: (b, 0)),
        out_shape=jax.ShapeDtypeStruct((_N, _NC), jnp.float32),
    )(pos, post)

    compact = functools.partial(
        pl.kernel,
        mesh=plsc.VectorSubcoreMesh(core_axis_name="c", subcore_axis_name="s"),
        compiler_params=pltpu.CompilerParams(needs_layout_passes=False),
        out_type=[
            jax.ShapeDtypeStruct((_EP,), jnp.int32),
            jax.ShapeDtypeStruct((_EP,), jnp.int32),
        ],
        scratch_types=[
            pltpu.VMEM((_NC,), jnp.float32),
            pltpu.VMEM((_SLOT,), jnp.int32),
            pltpu.VMEM((_SLOT,), jnp.int32),
            pltpu.SemaphoreType.DMA,
        ],
    )(_compact_kernel)
    src, dst = compact(maskf)

    w1 = jnp.zeros((_HID, 2 * _HID), jnp.float32)
    w1 = w1.at[:_NB, :_HID].set(W1k / math.sqrt(_NB))
    w1 = w1.at[:_NB, _HID:].set(W1v / math.sqrt(_NB))
    bk = W2k.reshape(_HID, _DIN, _DK).transpose(1, 0, 2).reshape(_DIN, _HID * _DK)
    bv = W2v.reshape(_HID, _DIN, _DOUT).transpose(1, 0, 2).reshape(_DIN, _HID * _DOUT)
    wbig = jnp.concatenate([bk, bv], axis=1) / (math.sqrt(_HID) * math.sqrt(_DIN))
    wqd = (Wq @ Wd) / (math.sqrt(_DIN) * math.sqrt(_DQ * _DK))

    qd = pl.pallas_call(
        _prep_kernel,
        grid=(_N // _BN,),
        in_specs=[
            pl.BlockSpec((_BN, _DIN), lambda b: (b, 0)),
            pl.BlockSpec((_DIN, _DK), lambda b: (0, 0)),
        ],
        out_specs=pl.BlockSpec((_BN, _DK), lambda b: (b, 0)),
        out_shape=jax.ShapeDtypeStruct((_N, _DK), jnp.float32),
    )(x, wqd)

    table_s = jnp.concatenate([x, pos, jnp.zeros((_N, 13), jnp.float32)], axis=1)
    table_d = jnp.concatenate([qd, pos, jnp.zeros((_N, 13), jnp.float32)], axis=1)
    gs = jnp.take(table_s, src, axis=0)
    gd = jnp.take(table_d, dst, axis=0)

    contrib = pl.pallas_call(
        _edge_kernel,
        grid=(_EP // _BE,),
        in_specs=[
            pl.BlockSpec((_BE, 48), lambda b: (b, 0)),
            pl.BlockSpec((_BE, 32), lambda b: (b, 0)),
            pl.BlockSpec((_HID, 2 * _HID), lambda b: (0, 0)),
            pl.BlockSpec((_DIN, _HID * (_DK + _DOUT)), lambda b: (0, 0)),
        ],
        out_specs=pl.BlockSpec((_BE, 64), lambda b: (b, 0)),
        out_shape=jax.ShapeDtypeStruct((_EP, 64), jnp.float32),
    )(gs, gd, w1, wbig)

    acc = jax.ops.segment_sum(contrib, dst, num_segments=_N)
    parts = acc[None]

    out = pl.pallas_call(
        _finalize_kernel,
        grid=(_N // _BN,),
        in_specs=[pl.BlockSpec((1, _BN, 64), lambda b: (0, b, 0))],
        out_specs=pl.BlockSpec((_BN, _DOUT), lambda b: (b, 0)),
        out_shape=jax.ShapeDtypeStruct((_N, _DOUT), jnp.float32),
    )(parts)
    return out
